# baseline (speedup 1.0000x reference)
import functools

import jax
import jax.numpy as jnp
from jax import lax
from jax.experimental import pallas as pl
from jax.experimental.pallas import tpu as pltpu
from jax.experimental.pallas import tpu_sc as plsc

N = 10000
D = 128
DH = 64
E = 320000
NP = 10240
NC = 2
NS = 16
K = 128
CH = 160
EP = NS * CH * K
SLAB = NP // NS

_mesh = plsc.VectorSubcoreMesh(core_axis_name="c", subcore_axis_name="s")


@functools.partial(
    pl.kernel, mesh=_mesh,
    compiler_params=pltpu.CompilerParams(use_tc_tiling_on_sc=False),
    out_type=jax.ShapeDtypeStruct((NC, NP, 16), jnp.float32),
    scratch_types=[
        pltpu.VMEM((CH // NC, K), jnp.int32),
        pltpu.VMEM((K, 16), jnp.float32),
        pltpu.VMEM_SHARED((NP, 16), jnp.float32),
    ],
)
def _sc_deg(dst_hbm, ones_hbm, zeros_hbm, out_hbm, dst_v, ones_v, accum):
    cid = lax.axis_index("c")
    sid = lax.axis_index("s")
    chw = CH // NC
    pltpu.sync_copy(dst_hbm.at[sid, pl.ds(cid * chw, chw)], dst_v)
    pltpu.sync_copy(ones_hbm, ones_v)
    pltpu.sync_copy(zeros_hbm.at[pl.ds(sid * SLAB, SLAB)],
                    accum.at[pl.ds(sid * SLAB, SLAB)])
    plsc.subcore_barrier()

    def body(ch, _):
        pltpu.sync_copy(ones_v, accum.at[dst_v.at[ch]], add=True)
        return 0

    lax.fori_loop(0, chw, body, 0)
    plsc.subcore_barrier()
    pltpu.sync_copy(accum.at[pl.ds(sid * SLAB, SLAB)],
                    out_hbm.at[cid, pl.ds(sid * SLAB, SLAB)])


@functools.partial(
    pl.kernel, mesh=_mesh,
    compiler_params=pltpu.CompilerParams(use_tc_tiling_on_sc=False),
    out_type=jax.ShapeDtypeStruct((NP, NC, DH), jnp.float32),
    scratch_types=[
        pltpu.VMEM((CH, K), jnp.int32),
        pltpu.VMEM((CH, K), jnp.int32),
        pltpu.VMEM((4, K, DH), jnp.float32),
        pltpu.VMEM_SHARED((NP, DH), jnp.float32),
        pltpu.SemaphoreType.DMA,
        pltpu.SemaphoreType.DMA,
    ],
)
def _sc_agg(src_hbm, dst_hbm, table_hbm, zeros_hbm, out_hbm,
            src_v, dst_v, rows_v, accum, gsem, ssem):
    cid = lax.axis_index("c")
    sid = lax.axis_index("s")
    tbl = table_hbm.at[cid]
    pltpu.sync_copy(src_hbm.at[sid], src_v)
    pltpu.sync_copy(dst_hbm.at[sid], dst_v)
    pltpu.sync_copy(zeros_hbm.at[pl.ds(sid * SLAB, SLAB)],
                    accum.at[pl.ds(sid * SLAB, SLAB)])
    plsc.subcore_barrier()

    for p in range(2):
        pltpu.async_copy(tbl.at[src_v.at[p]], rows_v.at[p], gsem)

    def body(g, _):
        for b in range(4):
            ch = 4 * g + b
            pltpu.make_async_copy(
                tbl.at[src_v.at[ch]], rows_v.at[b], gsem).wait()
            pltpu.async_copy(rows_v.at[b], accum.at[dst_v.at[ch]], ssem,
                             add=True)
            @pl.when(ch >= 2)
            def _():
                pltpu.make_async_copy(
                    rows_v.at[(b + 2) % 4], accum.at[dst_v.at[0]],
                    ssem).wait()

            @pl.when(ch + 2 < CH)
            def _():
                pltpu.async_copy(
                    tbl.at[src_v.at[ch + 2]], rows_v.at[(b + 2) % 4], gsem)
        return 0

    lax.fori_loop(0, CH // 4, body, 0)
    for p in range(2):
        pltpu.make_async_copy(rows_v.at[2 + p], accum.at[dst_v.at[0]],
                              ssem).wait()
    plsc.subcore_barrier()
    pltpu.sync_copy(accum.at[pl.ds(sid * SLAB, SLAB)],
                    out_hbm.at[pl.ds(sid * SLAB, SLAB), cid])


def _r_of(degp_blk):
    deg = jnp.sum(degp_blk, axis=(0, 2)) * (1.0 / 16.0)
    return lax.rsqrt(jnp.maximum(deg, 1.0))


def _tc0_body(degp_ref, x_ref, w_ref, v_ref):
    r = _r_of(degp_ref[...])
    u = jnp.dot(x_ref[...], w_ref[0], preferred_element_type=jnp.float32)
    v_ref[0] = u * r[:, None]


def _tc1_body(degp_ref, t_ref, w_ref, b_ref, v_ref):
    r = _r_of(degp_ref[...])
    b = b_ref[...]
    h0 = jax.nn.relu(t_ref[:, 0, :] * r[:, None] + b[:, :DH])
    h1 = jax.nn.relu(t_ref[:, 1, :] * r[:, None] + b[:, DH:])
    u = (jnp.dot(h0, w_ref[0, 0], preferred_element_type=jnp.float32)
         + jnp.dot(h1, w_ref[0, 1], preferred_element_type=jnp.float32))
    v_ref[0] = u * r[:, None]


def _tc2_body(degp_ref, t_ref, b_ref, z_ref):
    r = _r_of(degp_ref[...])
    b = b_ref[...]
    z0 = t_ref[:, 0, :] * r[:, None] + b[:, :DH]
    z1 = t_ref[:, 1, :] * r[:, None] + b[:, DH:]
    z_ref[...] = jnp.concatenate([z0, z1], axis=1)


def _tc0(degp, x, W0, BL=1000):
    return pl.pallas_call(
        _tc0_body,
        grid=(N // BL, NC),
        in_specs=[
            pl.BlockSpec((2, BL, 16), lambda i, j: (0, i, 0)),
            pl.BlockSpec((BL, D), lambda i, j: (i, 0)),
            pl.BlockSpec((1, D, DH), lambda i, j: (j, 0, 0)),
        ],
        out_specs=pl.BlockSpec((1, BL, DH), lambda i, j: (j, i, 0)),
        out_shape=jax.ShapeDtypeStruct((NC, NP, DH), jnp.float32),
    )(degp, x, W0.reshape(D, NC, DH).transpose(1, 0, 2))


def _tc1(degp, t, W1, b0, BL=1024):
    W1q = W1.reshape(NC, DH, NC, DH).transpose(2, 0, 1, 3)
    return pl.pallas_call(
        _tc1_body,
        grid=(NP // BL, NC),
        in_specs=[
            pl.BlockSpec((2, BL, 16), lambda i, j: (0, i, 0)),
            pl.BlockSpec((BL, NC, DH), lambda i, j: (i, 0, 0)),
            pl.BlockSpec((1, NC, DH, DH), lambda i, j: (j, 0, 0, 0)),
            pl.BlockSpec((1, D), lambda i, j: (0, 0)),
        ],
        out_specs=pl.BlockSpec((1, BL, DH), lambda i, j: (j, i, 0)),
        out_shape=jax.ShapeDtypeStruct((NC, NP, DH), jnp.float32),
    )(degp, t, W1q, b0)


def _tc2(degp, t, b1, BL=1000):
    return pl.pallas_call(
        _tc2_body,
        grid=(N // BL,),
        in_specs=[
            pl.BlockSpec((2, BL, 16), lambda i: (0, i, 0)),
            pl.BlockSpec((BL, NC, DH), lambda i: (i, 0, 0)),
            pl.BlockSpec((1, D), lambda i: (0, 0)),
        ],
        out_specs=pl.BlockSpec((BL, D), lambda i: (i, 0)),
        out_shape=jax.ShapeDtypeStruct((N, D), jnp.float32),
    )(degp, t, b1)


def kernel(x, ei, W0, b0, W1, b1):
    src = ei[0]
    dst = ei[1]
    pad = jnp.full((EP - E,), N, dtype=jnp.int32)
    srcp = jnp.concatenate([src, pad]).reshape(NS, CH, K)
    dstp = jnp.concatenate([dst, pad]).reshape(NS, CH, K)
    zeros64 = jnp.zeros((NP, DH), jnp.float32)
    zeros16 = jnp.zeros((NP, 16), jnp.float32)
    ones16 = jnp.ones((K, 16), jnp.float32)

    degp = _sc_deg(dstp, ones16, zeros16)
    v0 = _tc0(degp, x, W0)
    t0 = _sc_agg(srcp, dstp, v0, zeros64)
    v1 = _tc1(degp, t0, W1.reshape(NC, DH, D), b0.reshape(1, D))
    t1 = _sc_agg(srcp, dstp, v1, zeros64)
    return _tc2(degp, t1, b1.reshape(1, D))


# speedup vs baseline: 1.0971x; 1.0971x over previous
import functools

import jax
import jax.numpy as jnp
from jax import lax
from jax.experimental import pallas as pl
from jax.experimental.pallas import tpu as pltpu
from jax.experimental.pallas import tpu_sc as plsc

N = 10000
D = 128
DH = 64
E = 320000
NP = 10240
NC = 2
NS = 16
K = 128
CH = 160
EP = NS * CH * K
SLAB = NP // NS

_mesh = plsc.VectorSubcoreMesh(core_axis_name="c", subcore_axis_name="s")


@functools.partial(
    pl.kernel, mesh=_mesh,
    compiler_params=pltpu.CompilerParams(use_tc_tiling_on_sc=False),
    out_type=jax.ShapeDtypeStruct((NC, NP, 16), jnp.float32),
    scratch_types=[
        pltpu.VMEM((CH // NC, K), jnp.int32),
        pltpu.VMEM((K, 16), jnp.float32),
        pltpu.VMEM_SHARED((NP, 16), jnp.float32),
    ],
)
def _sc_deg(dst_hbm, ones_hbm, zeros_hbm, out_hbm, dst_v, ones_v, accum):
    cid = lax.axis_index("c")
    sid = lax.axis_index("s")
    chw = CH // NC
    pltpu.sync_copy(dst_hbm.at[sid, pl.ds(cid * chw, chw)], dst_v)
    pltpu.sync_copy(ones_hbm, ones_v)
    pltpu.sync_copy(zeros_hbm.at[pl.ds(sid * SLAB, SLAB)],
                    accum.at[pl.ds(sid * SLAB, SLAB)])
    plsc.subcore_barrier()

    def body(ch, _):
        pltpu.sync_copy(ones_v, accum.at[dst_v.at[ch]], add=True)
        return 0

    lax.fori_loop(0, chw, body, 0)
    plsc.subcore_barrier()
    pltpu.sync_copy(accum.at[pl.ds(sid * SLAB, SLAB)],
                    out_hbm.at[cid, pl.ds(sid * SLAB, SLAB)])


@functools.partial(
    pl.kernel, mesh=_mesh,
    compiler_params=pltpu.CompilerParams(use_tc_tiling_on_sc=False),
    out_type=jax.ShapeDtypeStruct((NP, NC, DH), jnp.float32),
    scratch_types=[
        pltpu.VMEM((CH, K), jnp.int32),
        pltpu.VMEM((CH, K), jnp.int32),
        pltpu.VMEM((4, K, DH), jnp.float32),
        pltpu.VMEM_SHARED((NP, DH), jnp.float32),
        pltpu.SemaphoreType.DMA,
        pltpu.SemaphoreType.DMA,
    ],
)
def _sc_agg(src_hbm, dst_hbm, table_hbm, zeros_hbm, out_hbm,
            src_v, dst_v, rows_v, accum, gsem, ssem):
    cid = lax.axis_index("c")
    sid = lax.axis_index("s")
    tbl = table_hbm.at[cid]
    pltpu.sync_copy(src_hbm.at[sid], src_v)
    pltpu.sync_copy(dst_hbm.at[sid], dst_v)
    pltpu.sync_copy(zeros_hbm.at[pl.ds(sid * SLAB, SLAB)],
                    accum.at[pl.ds(sid * SLAB, SLAB)])
    plsc.subcore_barrier()

    for p in range(2):
        pltpu.async_copy(tbl.at[src_v.at[p]], rows_v.at[p], gsem)

    def body(g, _):
        for b in range(4):
            ch = 4 * g + b
            pltpu.make_async_copy(
                tbl.at[src_v.at[ch]], rows_v.at[b], gsem).wait()
            pltpu.async_copy(rows_v.at[b], accum.at[dst_v.at[ch]], ssem,
                             add=True)
            @pl.when(ch >= 2)
            def _():
                pltpu.make_async_copy(
                    rows_v.at[(b + 2) % 4], accum.at[dst_v.at[0]],
                    ssem).wait()

            @pl.when(ch + 2 < CH)
            def _():
                pltpu.async_copy(
                    tbl.at[src_v.at[ch + 2]], rows_v.at[(b + 2) % 4], gsem)
        return 0

    lax.fori_loop(0, CH // 4, body, 0)
    for p in range(2):
        pltpu.make_async_copy(rows_v.at[2 + p], accum.at[dst_v.at[0]],
                              ssem).wait()
    plsc.subcore_barrier()
    pltpu.sync_copy(accum.at[pl.ds(sid * SLAB, SLAB)],
                    out_hbm.at[pl.ds(sid * SLAB, SLAB), cid])


def _r_of(degp_blk):
    deg = jnp.sum(degp_blk, axis=(0, 2)) * (1.0 / 16.0)
    return lax.rsqrt(jnp.maximum(deg, 1.0))


def _tc0_body(degp_ref, x_ref, w_ref, v_ref):
    r = _r_of(degp_ref[...])
    u = jnp.dot(x_ref[...], w_ref[0], preferred_element_type=jnp.float32)
    v_ref[0] = u * r[:, None]


def _tc1_body(degp_ref, t_ref, w_ref, b_ref, v_ref):
    r = _r_of(degp_ref[...])
    t = t_ref[...].reshape(t_ref.shape[0] * 8, D)
    h = jax.nn.relu(t * r[:, None] + b_ref[...])
    u = jnp.dot(h, w_ref[0], preferred_element_type=jnp.float32)
    v_ref[0] = u * r[:, None]


def _tc2_body(degp_ref, t_ref, b_ref, z_ref):
    r = _r_of(degp_ref[...])
    t = t_ref[...].reshape(t_ref.shape[0] * 8, D)
    z_ref[...] = t * r[:, None] + b_ref[...]


def _tc0(degp, x, W0, BL=1000):
    return pl.pallas_call(
        _tc0_body,
        grid=(N // BL, NC),
        in_specs=[
            pl.BlockSpec((2, BL, 16), lambda i, j: (0, i, 0)),
            pl.BlockSpec((BL, D), lambda i, j: (i, 0)),
            pl.BlockSpec((1, D, DH), lambda i, j: (j, 0, 0)),
        ],
        out_specs=pl.BlockSpec((1, BL, DH), lambda i, j: (j, i, 0)),
        out_shape=jax.ShapeDtypeStruct((NC, NP, DH), jnp.float32),
    )(degp, x, W0.reshape(D, NC, DH).transpose(1, 0, 2))


def _tc1(degp, t, W1, b0, BL=1024):
    tq = t.reshape(NP // 8, 8, D)
    W1r = W1.reshape(D, NC, DH).transpose(1, 0, 2)
    return pl.pallas_call(
        _tc1_body,
        grid=(NP // BL, NC),
        in_specs=[
            pl.BlockSpec((2, BL, 16), lambda i, j: (0, i, 0)),
            pl.BlockSpec((BL // 8, 8, D), lambda i, j: (i, 0, 0)),
            pl.BlockSpec((1, D, DH), lambda i, j: (j, 0, 0)),
            pl.BlockSpec((1, D), lambda i, j: (0, 0)),
        ],
        out_specs=pl.BlockSpec((1, BL, DH), lambda i, j: (j, i, 0)),
        out_shape=jax.ShapeDtypeStruct((NC, NP, DH), jnp.float32),
    )(degp, tq, W1r, b0)


def _tc2(degp, t, b1, BL=1000):
    tq = t.reshape(NP // 8, 8, D)
    return pl.pallas_call(
        _tc2_body,
        grid=(N // BL,),
        in_specs=[
            pl.BlockSpec((2, BL, 16), lambda i: (0, i, 0)),
            pl.BlockSpec((BL // 8, 8, D), lambda i: (i, 0, 0)),
            pl.BlockSpec((1, D), lambda i: (0, 0)),
        ],
        out_specs=pl.BlockSpec((BL, D), lambda i: (i, 0)),
        out_shape=jax.ShapeDtypeStruct((N, D), jnp.float32),
    )(degp, tq, b1)


def kernel(x, ei, W0, b0, W1, b1):
    src = ei[0]
    dst = ei[1]
    pad = jnp.full((EP - E,), N, dtype=jnp.int32)
    srcp = jnp.concatenate([src, pad]).reshape(NS, CH, K)
    dstp = jnp.concatenate([dst, pad]).reshape(NS, CH, K)
    zeros64 = jnp.zeros((NP, DH), jnp.float32)
    zeros16 = jnp.zeros((NP, 16), jnp.float32)
    ones16 = jnp.ones((K, 16), jnp.float32)

    degp = _sc_deg(dstp, ones16, zeros16)
    v0 = _tc0(degp, x, W0)
    t0 = _sc_agg(srcp, dstp, v0, zeros64)
    v1 = _tc1(degp, t0, W1.reshape(NC, DH, D), b0.reshape(1, D))
    t1 = _sc_agg(srcp, dstp, v1, zeros64)
    return _tc2(degp, t1, b1.reshape(1, D))


# speedup vs baseline: 1.1072x; 1.0092x over previous
import functools

import jax
import jax.numpy as jnp
from jax import lax
from jax.experimental import pallas as pl
from jax.experimental.pallas import tpu as pltpu
from jax.experimental.pallas import tpu_sc as plsc

N = 10000
D = 128
DH = 64
E = 320000
NP = 10240
NC = 2
NS = 16
K = 128
CH = 160
EP = NS * CH * K
SLAB = NP // NS

_mesh = plsc.VectorSubcoreMesh(core_axis_name="c", subcore_axis_name="s")


@functools.partial(
    pl.kernel, mesh=_mesh,
    compiler_params=pltpu.CompilerParams(use_tc_tiling_on_sc=False),
    out_type=jax.ShapeDtypeStruct((NC, NP, 16), jnp.float32),
    scratch_types=[
        pltpu.VMEM((CH // NC, K), jnp.int32),
        pltpu.VMEM((K, 16), jnp.float32),
        pltpu.VMEM_SHARED((NP, 16), jnp.float32),
    ],
)
def _sc_deg(dst_hbm, ones_hbm, zeros_hbm, out_hbm, dst_v, ones_v, accum):
    cid = lax.axis_index("c")
    sid = lax.axis_index("s")
    chw = CH // NC
    pltpu.sync_copy(dst_hbm.at[sid, pl.ds(cid * chw, chw)], dst_v)
    pltpu.sync_copy(ones_hbm, ones_v)
    pltpu.sync_copy(zeros_hbm.at[pl.ds(sid * SLAB, SLAB)],
                    accum.at[pl.ds(sid * SLAB, SLAB)])
    plsc.subcore_barrier()

    def body(ch, _):
        pltpu.sync_copy(ones_v, accum.at[dst_v.at[ch]], add=True)
        return 0

    lax.fori_loop(0, chw, body, 0)
    plsc.subcore_barrier()
    pltpu.sync_copy(accum.at[pl.ds(sid * SLAB, SLAB)],
                    out_hbm.at[cid, pl.ds(sid * SLAB, SLAB)])


@functools.partial(
    pl.kernel, mesh=_mesh,
    compiler_params=pltpu.CompilerParams(use_tc_tiling_on_sc=False),
    out_type=jax.ShapeDtypeStruct((NP, NC, DH), jnp.float32),
    scratch_types=[
        pltpu.VMEM((CH, K), jnp.int32),
        pltpu.VMEM((CH, K), jnp.int32),
        pltpu.VMEM((4, K, DH), jnp.float32),
        pltpu.VMEM_SHARED((NP, DH), jnp.float32),
        pltpu.SemaphoreType.DMA,
        pltpu.SemaphoreType.DMA,
    ],
)
def _sc_agg(src_hbm, dst_hbm, table_hbm, zeros_hbm, out_hbm,
            src_v, dst_v, rows_v, accum, gsem, ssem):
    cid = lax.axis_index("c")
    sid = lax.axis_index("s")
    tbl = table_hbm.at[cid]
    pltpu.sync_copy(src_hbm.at[sid], src_v)
    pltpu.sync_copy(dst_hbm.at[sid], dst_v)
    pltpu.sync_copy(zeros_hbm.at[pl.ds(sid * SLAB, SLAB)],
                    accum.at[pl.ds(sid * SLAB, SLAB)])
    plsc.subcore_barrier()

    for p in range(2):
        pltpu.async_copy(tbl.at[src_v.at[p]], rows_v.at[p], gsem)

    def body(g, _):
        for b in range(4):
            ch = 4 * g + b
            pltpu.make_async_copy(
                tbl.at[src_v.at[ch]], rows_v.at[b], gsem).wait()
            pltpu.async_copy(rows_v.at[b], accum.at[dst_v.at[ch]], ssem,
                             add=True)
            @pl.when(ch >= 2)
            def _():
                pltpu.make_async_copy(
                    rows_v.at[(b + 2) % 4], accum.at[dst_v.at[0]],
                    ssem).wait()

            @pl.when(ch + 2 < CH)
            def _():
                pltpu.async_copy(
                    tbl.at[src_v.at[ch + 2]], rows_v.at[(b + 2) % 4], gsem)
        return 0

    lax.fori_loop(0, CH // 4, body, 0)
    for p in range(2):
        pltpu.make_async_copy(rows_v.at[2 + p], accum.at[dst_v.at[0]],
                              ssem).wait()
    plsc.subcore_barrier()
    pltpu.sync_copy(accum.at[pl.ds(sid * SLAB, SLAB)],
                    out_hbm.at[pl.ds(sid * SLAB, SLAB), cid])


def _r_of(degp_blk):
    deg = jnp.sum(degp_blk, axis=(0, 2)) * (1.0 / 16.0)
    return lax.rsqrt(jnp.maximum(deg, 1.0))


def _tc0_body(degp_ref, x_ref, w_ref, v_ref):
    r = _r_of(degp_ref[...])
    u = jnp.dot(x_ref[...], w_ref[0], preferred_element_type=jnp.float32)
    v_ref[0] = u * r[:, None]


def _tc1_body(degp_ref, t_ref, w_ref, b_ref, v_ref):
    r = _r_of(degp_ref[...])
    t = t_ref[...].reshape(t_ref.shape[0] * 8, D)
    h = jax.nn.relu(t * r[:, None] + b_ref[...])
    u = jnp.dot(h, w_ref[0], preferred_element_type=jnp.float32)
    v_ref[0] = u * r[:, None]


def _tc2_body(degp_ref, t_ref, b_ref, z_ref):
    r = _r_of(degp_ref[...])
    t = t_ref[...].reshape(t_ref.shape[0] * 8, D)
    z_ref[...] = t * r[:, None] + b_ref[...]


def _tc0(degp, x, W0, BL=2000):
    return pl.pallas_call(
        _tc0_body,
        grid=(N // BL, NC),
        in_specs=[
            pl.BlockSpec((2, BL, 16), lambda i, j: (0, i, 0)),
            pl.BlockSpec((BL, D), lambda i, j: (i, 0)),
            pl.BlockSpec((1, D, DH), lambda i, j: (j, 0, 0)),
        ],
        out_specs=pl.BlockSpec((1, BL, DH), lambda i, j: (j, i, 0)),
        out_shape=jax.ShapeDtypeStruct((NC, NP, DH), jnp.float32),
    )(degp, x, W0.reshape(D, NC, DH).transpose(1, 0, 2))


def _tc1(degp, t, W1, b0, BL=1024):
    tq = t.reshape(NP // 8, 8, D)
    W1r = W1.reshape(D, NC, DH).transpose(1, 0, 2)
    return pl.pallas_call(
        _tc1_body,
        grid=(NP // BL, NC),
        in_specs=[
            pl.BlockSpec((2, BL, 16), lambda i, j: (0, i, 0)),
            pl.BlockSpec((BL // 8, 8, D), lambda i, j: (i, 0, 0)),
            pl.BlockSpec((1, D, DH), lambda i, j: (j, 0, 0)),
            pl.BlockSpec((1, D), lambda i, j: (0, 0)),
        ],
        out_specs=pl.BlockSpec((1, BL, DH), lambda i, j: (j, i, 0)),
        out_shape=jax.ShapeDtypeStruct((NC, NP, DH), jnp.float32),
    )(degp, tq, W1r, b0)


def _tc2(degp, t, b1, BL=1000):
    tq = t.reshape(NP // 8, 8, D)
    return pl.pallas_call(
        _tc2_body,
        grid=(N // BL,),
        in_specs=[
            pl.BlockSpec((2, BL, 16), lambda i: (0, i, 0)),
            pl.BlockSpec((BL // 8, 8, D), lambda i: (i, 0, 0)),
            pl.BlockSpec((1, D), lambda i: (0, 0)),
        ],
        out_specs=pl.BlockSpec((BL, D), lambda i: (i, 0)),
        out_shape=jax.ShapeDtypeStruct((N, D), jnp.float32),
    )(degp, tq, b1)


def kernel(x, ei, W0, b0, W1, b1):
    src = ei[0]
    dst = ei[1]
    pad = jnp.full((EP - E,), N, dtype=jnp.int32)
    srcp = jnp.concatenate([src, pad]).reshape(NS, CH, K)
    dstp = jnp.concatenate([dst, pad]).reshape(NS, CH, K)
    zeros64 = jnp.zeros((NP, DH), jnp.float32)
    zeros16 = jnp.zeros((NP, 16), jnp.float32)
    ones16 = jnp.ones((K, 16), jnp.float32)

    degp = _sc_deg(dstp, ones16, zeros16)
    v0 = _tc0(degp, x, W0)
    t0 = _sc_agg(srcp, dstp, v0, zeros64)
    v1 = _tc1(degp, t0, W1, b0.reshape(1, D))
    t1 = _sc_agg(srcp, dstp, v1, zeros64)
    return _tc2(degp, t1, b1.reshape(1, D))


# speedup vs baseline: 1.1724x; 1.0588x over previous
import functools

import jax
import jax.numpy as jnp
from jax import lax
from jax.experimental import pallas as pl
from jax.experimental.pallas import tpu as pltpu
from jax.experimental.pallas import tpu_sc as plsc

N = 10000
D = 128
DH = 64
E = 320000
NP = 10240
NC = 2
NS = 16
K = 128
CH = 160
EP = NS * CH * K
SLAB = NP // NS

_mesh = plsc.VectorSubcoreMesh(core_axis_name="c", subcore_axis_name="s")


@functools.partial(
    pl.kernel, mesh=_mesh,
    compiler_params=pltpu.CompilerParams(use_tc_tiling_on_sc=False),
    out_type=jax.ShapeDtypeStruct((NC, NP, 16), jnp.float32),
    scratch_types=[
        pltpu.VMEM((CH // NC, K), jnp.int32),
        pltpu.VMEM((K, 16), jnp.float32),
        pltpu.VMEM_SHARED((NP, 16), jnp.float32),
    ],
)
def _sc_deg(dst_hbm, ones_hbm, zeros_hbm, out_hbm, dst_v, ones_v, accum):
    cid = lax.axis_index("c")
    sid = lax.axis_index("s")
    chw = CH // NC
    pltpu.sync_copy(dst_hbm.at[sid, pl.ds(cid * chw, chw)], dst_v)
    pltpu.sync_copy(ones_hbm, ones_v)
    pltpu.sync_copy(zeros_hbm.at[pl.ds(sid * SLAB, SLAB)],
                    accum.at[pl.ds(sid * SLAB, SLAB)])
    plsc.subcore_barrier()

    def body(ch, _):
        pltpu.sync_copy(ones_v, accum.at[dst_v.at[ch]], add=True)
        return 0

    lax.fori_loop(0, chw, body, 0)
    plsc.subcore_barrier()
    pltpu.sync_copy(accum.at[pl.ds(sid * SLAB, SLAB)],
                    out_hbm.at[cid, pl.ds(sid * SLAB, SLAB)])


@functools.partial(
    pl.kernel, mesh=_mesh,
    compiler_params=pltpu.CompilerParams(use_tc_tiling_on_sc=False),
    out_type=jax.ShapeDtypeStruct((NP, NC, DH), jnp.float32),
    scratch_types=[
        pltpu.VMEM((CH, K), jnp.int32),
        pltpu.VMEM((CH, K), jnp.int32),
        pltpu.VMEM((5, K, DH), jnp.float32),
        pltpu.VMEM_SHARED((NP, DH), jnp.float32),
        pltpu.SemaphoreType.DMA,
        pltpu.SemaphoreType.DMA,
    ],
)
def _sc_agg(src_hbm, dst_hbm, table_hbm, zeros_hbm, out_hbm,
            src_v, dst_v, rows_v, accum, gsem, ssem):
    cid = lax.axis_index("c")
    sid = lax.axis_index("s")
    tbl = table_hbm.at[cid]
    pltpu.sync_copy(src_hbm.at[sid], src_v)
    pltpu.sync_copy(dst_hbm.at[sid], dst_v)
    pltpu.sync_copy(zeros_hbm.at[pl.ds(sid * SLAB, SLAB)],
                    accum.at[pl.ds(sid * SLAB, SLAB)])
    plsc.subcore_barrier()

    for p in range(2):
        pltpu.async_copy(tbl.at[src_v.at[p]], rows_v.at[p], gsem)

    def body(g, _):
        for b in range(5):
            ch = 5 * g + b
            pltpu.make_async_copy(
                tbl.at[src_v.at[ch]], rows_v.at[b], gsem).wait()
            pltpu.async_copy(rows_v.at[b], accum.at[dst_v.at[ch]], ssem,
                             add=True)
            @pl.when(ch >= 3)
            def _():
                pltpu.make_async_copy(
                    rows_v.at[(b + 2) % 5], accum.at[dst_v.at[0]],
                    ssem).wait()

            @pl.when(ch + 2 < CH)
            def _():
                pltpu.async_copy(
                    tbl.at[src_v.at[ch + 2]], rows_v.at[(b + 2) % 5], gsem)
        return 0

    lax.fori_loop(0, CH // 5, body, 0)
    for p in range(3):
        pltpu.make_async_copy(rows_v.at[(157 + p) % 5], accum.at[dst_v.at[0]],
                              ssem).wait()
    plsc.subcore_barrier()
    pltpu.sync_copy(accum.at[pl.ds(sid * SLAB, SLAB)],
                    out_hbm.at[pl.ds(sid * SLAB, SLAB), cid])


def _r_of(degp_blk):
    deg = jnp.sum(degp_blk, axis=(0, 2)) * (1.0 / 16.0)
    return lax.rsqrt(jnp.maximum(deg, 1.0))


def _tc0_body(degp_ref, x_ref, w_ref, v_ref):
    r = _r_of(degp_ref[...])
    u = jnp.dot(x_ref[...], w_ref[...], preferred_element_type=jnp.float32)
    v = u * r[:, None]
    v_ref[0] = v[:, :DH]
    v_ref[1] = v[:, DH:]


def _tc1_body(degp_ref, t_ref, w_ref, b_ref, v_ref):
    r = _r_of(degp_ref[...])
    t = t_ref[...].reshape(t_ref.shape[0] * 8, D)
    h = jax.nn.relu(t * r[:, None] + b_ref[...])
    u = jnp.dot(h, w_ref[...], preferred_element_type=jnp.float32)
    v = u * r[:, None]
    v_ref[0] = v[:, :DH]
    v_ref[1] = v[:, DH:]


def _tc2_body(degp_ref, t_ref, b_ref, z_ref):
    r = _r_of(degp_ref[...])
    t = t_ref[...].reshape(t_ref.shape[0] * 8, D)
    z_ref[...] = t * r[:, None] + b_ref[...]


def _tc0(degp, x, W0, BL=2000):
    return pl.pallas_call(
        _tc0_body,
        grid=(N // BL,),
        in_specs=[
            pl.BlockSpec((2, BL, 16), lambda i: (0, i, 0)),
            pl.BlockSpec((BL, D), lambda i: (i, 0)),
            pl.BlockSpec((D, D), lambda i: (0, 0)),
        ],
        out_specs=pl.BlockSpec((NC, BL, DH), lambda i: (0, i, 0)),
        out_shape=jax.ShapeDtypeStruct((NC, NP, DH), jnp.float32),
    )(degp, x, W0)


def _tc1(degp, t, W1, b0, BL=1024):
    tq = t.reshape(NP // 8, 8, D)
    return pl.pallas_call(
        _tc1_body,
        grid=(NP // BL,),
        in_specs=[
            pl.BlockSpec((2, BL, 16), lambda i: (0, i, 0)),
            pl.BlockSpec((BL // 8, 8, D), lambda i: (i, 0, 0)),
            pl.BlockSpec((D, D), lambda i: (0, 0)),
            pl.BlockSpec((1, D), lambda i: (0, 0)),
        ],
        out_specs=pl.BlockSpec((NC, BL, DH), lambda i: (0, i, 0)),
        out_shape=jax.ShapeDtypeStruct((NC, NP, DH), jnp.float32),
    )(degp, tq, W1, b0)


def _tc2(degp, t, b1, BL=1000):
    tq = t.reshape(NP // 8, 8, D)
    return pl.pallas_call(
        _tc2_body,
        grid=(N // BL,),
        in_specs=[
            pl.BlockSpec((2, BL, 16), lambda i: (0, i, 0)),
            pl.BlockSpec((BL // 8, 8, D), lambda i: (i, 0, 0)),
            pl.BlockSpec((1, D), lambda i: (0, 0)),
        ],
        out_specs=pl.BlockSpec((BL, D), lambda i: (i, 0)),
        out_shape=jax.ShapeDtypeStruct((N, D), jnp.float32),
    )(degp, tq, b1)


def kernel(x, ei, W0, b0, W1, b1):
    src = ei[0]
    dst = ei[1]
    pad = jnp.full((EP - E,), N, dtype=jnp.int32)
    srcp = jnp.concatenate([src, pad]).reshape(NS, CH, K)
    dstp = jnp.concatenate([dst, pad]).reshape(NS, CH, K)
    zeros64 = jnp.zeros((NP, DH), jnp.float32)
    zeros16 = jnp.zeros((NP, 16), jnp.float32)
    ones16 = jnp.ones((K, 16), jnp.float32)

    degp = _sc_deg(dstp, ones16, zeros16)
    v0 = _tc0(degp, x, W0)
    t0 = _sc_agg(srcp, dstp, v0, zeros64)
    v1 = _tc1(degp, t0, W1, b0.reshape(1, D))
    t1 = _sc_agg(srcp, dstp, v1, zeros64)
    return _tc2(degp, t1, b1.reshape(1, D))


# speedup vs baseline: 1.1895x; 1.0146x over previous
import functools

import jax
import jax.numpy as jnp
from jax import lax
from jax.experimental import pallas as pl
from jax.experimental.pallas import tpu as pltpu
from jax.experimental.pallas import tpu_sc as plsc

N = 10000
D = 128
DH = 64
E = 320000
NP = 10240
NC = 2
NS = 16
K = 128
CH = 160
EP = NS * CH * K
SLAB = NP // NS

_mesh = plsc.VectorSubcoreMesh(core_axis_name="c", subcore_axis_name="s")


@functools.partial(
    pl.kernel, mesh=_mesh,
    compiler_params=pltpu.CompilerParams(use_tc_tiling_on_sc=False),
    out_type=jax.ShapeDtypeStruct((NC, NP, 16), jnp.float32),
    scratch_types=[
        pltpu.VMEM((CH // NC, K), jnp.int32),
        pltpu.VMEM((K, 16), jnp.float32),
        pltpu.VMEM_SHARED((NP, 16), jnp.float32),
    ],
)
def _sc_deg(dst_hbm, ones_hbm, zeros_hbm, out_hbm, dst_v, ones_v, accum):
    cid = lax.axis_index("c")
    sid = lax.axis_index("s")
    chw = CH // NC
    pltpu.sync_copy(dst_hbm.at[sid, pl.ds(cid * chw, chw)], dst_v)
    pltpu.sync_copy(ones_hbm, ones_v)
    pltpu.sync_copy(zeros_hbm.at[pl.ds(sid * SLAB, SLAB)],
                    accum.at[pl.ds(sid * SLAB, SLAB)])
    plsc.subcore_barrier()

    def body(ch, _):
        pltpu.sync_copy(ones_v, accum.at[dst_v.at[ch]], add=True)
        return 0

    lax.fori_loop(0, chw, body, 0)
    plsc.subcore_barrier()
    pltpu.sync_copy(accum.at[pl.ds(sid * SLAB, SLAB)],
                    out_hbm.at[cid, pl.ds(sid * SLAB, SLAB)])


@functools.partial(
    pl.kernel, mesh=_mesh,
    compiler_params=pltpu.CompilerParams(use_tc_tiling_on_sc=False),
    out_type=jax.ShapeDtypeStruct((NP, NC, DH), jnp.float32),
    scratch_types=[
        pltpu.VMEM((CH, K), jnp.int32),
        pltpu.VMEM((CH, K), jnp.int32),
        pltpu.VMEM((5, K, DH), jnp.float32),
        pltpu.VMEM_SHARED((NP, DH), jnp.float32),
        pltpu.SemaphoreType.DMA,
        pltpu.SemaphoreType.DMA,
    ],
)
def _sc_agg(src_hbm, dst_hbm, table_hbm, zeros_hbm, out_hbm,
            src_v, dst_v, rows_v, accum, gsem, ssem):
    cid = lax.axis_index("c")
    sid = lax.axis_index("s")
    tbl = table_hbm.at[cid]
    pltpu.sync_copy(src_hbm.at[sid], src_v)
    pltpu.sync_copy(dst_hbm.at[sid], dst_v)
    pltpu.sync_copy(zeros_hbm.at[pl.ds(sid * SLAB, SLAB)],
                    accum.at[pl.ds(sid * SLAB, SLAB)])
    plsc.subcore_barrier()

    for p in range(3):
        pltpu.async_copy(tbl.at[src_v.at[p]], rows_v.at[p], gsem)

    def body(g, _):
        for b in range(5):
            ch = 5 * g + b
            pltpu.make_async_copy(
                tbl.at[src_v.at[ch]], rows_v.at[b], gsem).wait()
            pltpu.async_copy(rows_v.at[b], accum.at[dst_v.at[ch]], ssem,
                             add=True)
            @pl.when(ch >= 2)
            def _():
                pltpu.make_async_copy(
                    rows_v.at[(b + 3) % 5], accum.at[dst_v.at[0]],
                    ssem).wait()

            @pl.when(ch + 3 < CH)
            def _():
                pltpu.async_copy(
                    tbl.at[src_v.at[ch + 3]], rows_v.at[(b + 3) % 5], gsem)
        return 0

    lax.fori_loop(0, CH // 5, body, 0)
    for p in range(2):
        pltpu.make_async_copy(rows_v.at[(158 + p) % 5], accum.at[dst_v.at[0]],
                              ssem).wait()
    plsc.subcore_barrier()
    pltpu.sync_copy(accum.at[pl.ds(sid * SLAB, SLAB)],
                    out_hbm.at[pl.ds(sid * SLAB, SLAB), cid])


def _r_of(degp_blk):
    deg = jnp.sum(degp_blk, axis=(0, 2)) * (1.0 / 16.0)
    return lax.rsqrt(jnp.maximum(deg, 1.0))


def _tc0_body(degp_ref, x_ref, w_ref, v_ref):
    r = _r_of(degp_ref[...])
    u = jnp.dot(x_ref[...], w_ref[...], preferred_element_type=jnp.float32)
    v = u * r[:, None]
    v_ref[0] = v[:, :DH]
    v_ref[1] = v[:, DH:]


def _tc1_body(degp_ref, t_ref, w_ref, b_ref, v_ref):
    r = _r_of(degp_ref[...])
    t = t_ref[...].reshape(t_ref.shape[0] * 8, D)
    h = jax.nn.relu(t * r[:, None] + b_ref[...])
    u = jnp.dot(h, w_ref[...], preferred_element_type=jnp.float32)
    v = u * r[:, None]
    v_ref[0] = v[:, :DH]
    v_ref[1] = v[:, DH:]


def _tc2_body(degp_ref, t_ref, b_ref, z_ref):
    r = _r_of(degp_ref[...])
    t = t_ref[...].reshape(t_ref.shape[0] * 8, D)
    z_ref[...] = t * r[:, None] + b_ref[...]


def _tc0(degp, x, W0, BL=2000):
    return pl.pallas_call(
        _tc0_body,
        grid=(N // BL,),
        in_specs=[
            pl.BlockSpec((2, BL, 16), lambda i: (0, i, 0)),
            pl.BlockSpec((BL, D), lambda i: (i, 0)),
            pl.BlockSpec((D, D), lambda i: (0, 0)),
        ],
        out_specs=pl.BlockSpec((NC, BL, DH), lambda i: (0, i, 0)),
        out_shape=jax.ShapeDtypeStruct((NC, NP, DH), jnp.float32),
    )(degp, x, W0)


def _tc1(degp, t, W1, b0, BL=1024):
    tq = t.reshape(NP // 8, 8, D)
    return pl.pallas_call(
        _tc1_body,
        grid=(NP // BL,),
        in_specs=[
            pl.BlockSpec((2, BL, 16), lambda i: (0, i, 0)),
            pl.BlockSpec((BL // 8, 8, D), lambda i: (i, 0, 0)),
            pl.BlockSpec((D, D), lambda i: (0, 0)),
            pl.BlockSpec((1, D), lambda i: (0, 0)),
        ],
        out_specs=pl.BlockSpec((NC, BL, DH), lambda i: (0, i, 0)),
        out_shape=jax.ShapeDtypeStruct((NC, NP, DH), jnp.float32),
    )(degp, tq, W1, b0)


def _tc2(degp, t, b1, BL=1000):
    tq = t.reshape(NP // 8, 8, D)
    return pl.pallas_call(
        _tc2_body,
        grid=(N // BL,),
        in_specs=[
            pl.BlockSpec((2, BL, 16), lambda i: (0, i, 0)),
            pl.BlockSpec((BL // 8, 8, D), lambda i: (i, 0, 0)),
            pl.BlockSpec((1, D), lambda i: (0, 0)),
        ],
        out_specs=pl.BlockSpec((BL, D), lambda i: (i, 0)),
        out_shape=jax.ShapeDtypeStruct((N, D), jnp.float32),
    )(degp, tq, b1)


def kernel(x, ei, W0, b0, W1, b1):
    src = ei[0]
    dst = ei[1]
    pad = jnp.full((EP - E,), N, dtype=jnp.int32)
    srcp = jnp.concatenate([src, pad]).reshape(NS, CH, K)
    dstp = jnp.concatenate([dst, pad]).reshape(NS, CH, K)
    zeros64 = jnp.zeros((NP, DH), jnp.float32)
    zeros16 = jnp.zeros((NP, 16), jnp.float32)
    ones16 = jnp.ones((K, 16), jnp.float32)

    degp = _sc_deg(dstp, ones16, zeros16)
    v0 = _tc0(degp, x, W0)
    t0 = _sc_agg(srcp, dstp, v0, zeros64)
    v1 = _tc1(degp, t0, W1, b0.reshape(1, D))
    t1 = _sc_agg(srcp, dstp, v1, zeros64)
    return _tc2(degp, t1, b1.reshape(1, D))
